# Initial kernel scaffold; baseline (speedup 1.0000x reference)
import jax
import jax.numpy as jnp
from jax.experimental import pallas as pl


def kernel(x, edge_index, W_gcn, b_gcn, W_ih, W_hh, b_ih, b_hh, W_cls, b_cls):
    raise NotImplementedError("write your pallas kernel here")


# speedup vs baseline: 89.9932x; 89.9932x over previous
import functools

import jax
import jax.numpy as jnp
from jax import lax
from jax.experimental import pallas as pl
from jax.experimental.pallas import tpu as pltpu
from jax.experimental.pallas import tpu_sc as plsc

T, N, F, H, C, E = 8, 50000, 16, 32, 5, 800000
NC, NS = 2, 16
TPC = T // NC
CH = 80
NCH = E // CH
GC = 25
CPT = NCH // NS
NG = CPT // GC
ROWS_PT = N // NS
ZROWS = 625
SEG = 3128
SEG_LAST = N - (NS - 1) * SEG

@functools.cache
def _sc_mesh():
    return plsc.VectorSubcoreMesh(core_axis_name="c", subcore_axis_name="s",
                                  num_cores=NC, num_subcores=NS)


def _deg_body(edges, ones_hbm, zeros_hbm, deg_out, idx_b, ones_v, zb, deg_sh,
              sem_s):
    c = lax.axis_index("c")
    s = lax.axis_index("s")
    pltpu.sync_copy(ones_hbm, ones_v)
    pltpu.sync_copy(zeros_hbm, zb)
    for tl in range(TPC):
        t = c * TPC + tl
        @pl.when(s < NS - 1)
        def _():
            pltpu.sync_copy(zb.at[pl.ds(0, SEG)], deg_sh.at[pl.ds(s * SEG, SEG)])
        @pl.when(s == NS - 1)
        def _():
            pltpu.sync_copy(zb.at[pl.ds(0, SEG_LAST)],
                            deg_sh.at[pl.ds((NS - 1) * SEG, SEG_LAST)])
        plsc.subcore_barrier()

        def group(g, carry):
            cb = s * CPT + g * GC
            pltpu.sync_copy(edges.at[t, 1, pl.ds(cb, GC)], idx_b)
            descs = []
            for k in range(GC):
                descs.append(pltpu.async_copy(
                    ones_v, deg_sh.at[idx_b.at[k]], sem_s, add=True))
            for d in descs:
                d.wait()
            return carry
        lax.fori_loop(0, NG, group, 0)
        plsc.subcore_barrier()

        @pl.when(s < NS - 1)
        def _():
            pltpu.sync_copy(deg_sh.at[pl.ds(s * SEG, SEG)],
                            deg_out.at[t, pl.ds(s * SEG, SEG)])
        @pl.when(s == NS - 1)
        def _():
            pltpu.sync_copy(deg_sh.at[pl.ds((NS - 1) * SEG, SEG_LAST)],
                            deg_out.at[t, pl.ds((NS - 1) * SEG, SEG_LAST)])
        plsc.subcore_barrier()


def _sc_degree(*args):
    return pl.kernel(
        _deg_body,
        out_type=jax.ShapeDtypeStruct((T, N), jnp.float32),
        mesh=_sc_mesh(),
        compiler_params=pltpu.CompilerParams(use_tc_tiling_on_sc=False),
        scratch_types=[
            pltpu.VMEM((GC, CH), jnp.int32),
            pltpu.VMEM((CH,), jnp.float32),
            pltpu.VMEM((SEG,), jnp.float32),
            pltpu.VMEM_SHARED((N,), jnp.float32),
            pltpu.SemaphoreType.DMA,
        ],
    )(*args)


def _prep_body(deg_ref, x_ref, dinv_ref, xd_ref):
    dg = deg_ref[...] + 1.0
    di = lax.rsqrt(dg)
    dinv_ref[...] = di
    xd_ref[...] = x_ref[...] * di[..., None]


def _tc_prep(deg_r, x_r):
    nrows, ncols = deg_r.shape
    bb = 8
    return pl.pallas_call(
        _prep_body,
        grid=(nrows // bb,),
        in_specs=[
            pl.BlockSpec((bb, ncols), lambda i: (i, 0)),
            pl.BlockSpec((bb, ncols, F), lambda i: (i, 0, 0)),
        ],
        out_specs=[
            pl.BlockSpec((bb, ncols), lambda i: (i, 0)),
            pl.BlockSpec((bb, ncols, F), lambda i: (i, 0, 0)),
        ],
        out_shape=[
            jax.ShapeDtypeStruct((nrows, ncols), jnp.float32),
            jax.ShapeDtypeStruct((nrows, ncols, F), jnp.float32),
        ],
    )(deg_r, x_r)


def _agg_body(edges, xd_flat, zeros2_hbm, acc_out, sidx, didx, rows, zb2,
              acc_sh, sem_g, sem_s):
    c = lax.axis_index("c")
    s = lax.axis_index("s")
    pltpu.sync_copy(zeros2_hbm, zb2)
    for tl in range(TPC):
        t = c * TPC + tl
        tN = t * N
        for z in range(ROWS_PT // ZROWS):
            pltpu.sync_copy(
                zb2, acc_sh.at[pl.ds(s * ROWS_PT + z * ZROWS, ZROWS)])
        plsc.subcore_barrier()

        def group(g, carry):
            cb = s * CPT + g * GC
            pltpu.sync_copy(edges.at[t, 0, pl.ds(cb, GC)], sidx)
            pltpu.sync_copy(edges.at[t, 1, pl.ds(cb, GC)], didx)
            for k in range(GC):
                for j in range(CH // 16):
                    v = sidx[k, pl.ds(j * 16, 16)]
                    sidx[k, pl.ds(j * 16, 16)] = v + tN
            gd = []
            for k in range(GC):
                gd.append(pltpu.async_copy(
                    xd_flat.at[sidx.at[k]], rows.at[k], sem_g))
            sd = []
            for k in range(GC):
                gd[k].wait()
                sd.append(pltpu.async_copy(
                    rows.at[k], acc_sh.at[didx.at[k]], sem_s, add=True))
            for d in sd:
                d.wait()
            return carry
        lax.fori_loop(0, NG, group, 0)
        plsc.subcore_barrier()

        for z in range(ROWS_PT // ZROWS):
            r0 = s * ROWS_PT + z * ZROWS
            pltpu.sync_copy(acc_sh.at[pl.ds(r0, ZROWS)],
                            acc_out.at[t, pl.ds(r0, ZROWS)])
        plsc.subcore_barrier()


def _sc_aggregate(*args):
    return pl.kernel(
        _agg_body,
        out_type=jax.ShapeDtypeStruct((T, N, F), jnp.float32),
        mesh=_sc_mesh(),
        compiler_params=pltpu.CompilerParams(use_tc_tiling_on_sc=False),
        scratch_types=[
            pltpu.VMEM((GC, CH), jnp.int32),
            pltpu.VMEM((GC, CH), jnp.int32),
            pltpu.VMEM((GC, CH, F), jnp.float32),
            pltpu.VMEM((ZROWS, F), jnp.float32),
            pltpu.VMEM_SHARED((N, F), jnp.float32),
            pltpu.SemaphoreType.DMA,
            pltpu.SemaphoreType.DMA,
        ],
    )(*args)


def _pool_body(acc_ref, x_ref, dinv_ref, w_ref, b_ref, out_ref):
    i = pl.program_id(1)
    a = acc_ref[0]
    xx = x_ref[0]
    di = dinv_ref[0]
    z = di * a + (di * di) * xx
    y = jnp.dot(z, w_ref[...], preferred_element_type=jnp.float32) + b_ref[...]
    y = jnp.maximum(y, 0.0)
    ps = jnp.sum(y, axis=0, keepdims=True)[None]

    @pl.when(i == 0)
    def _():
        out_ref[...] = ps

    @pl.when(i > 0)
    def _():
        out_ref[...] += ps


def _tc_pool(accF, x, dinv3, W, b2):
    br = 2000
    nb = N // br
    return pl.pallas_call(
        _pool_body,
        grid=(T, nb),
        in_specs=[
            pl.BlockSpec((1, br, F), lambda t, i: (t, i, 0)),
            pl.BlockSpec((1, br, F), lambda t, i: (t, i, 0)),
            pl.BlockSpec((1, br, 1), lambda t, i: (t, i, 0)),
            pl.BlockSpec((F, H), lambda t, i: (0, 0)),
            pl.BlockSpec((1, H), lambda t, i: (0, 0)),
        ],
        out_specs=pl.BlockSpec((1, 1, H), lambda t, i: (t, 0, 0)),
        out_shape=jax.ShapeDtypeStruct((T, 1, H), jnp.float32),
    )(accF, x, dinv3, W, b2).reshape(T, H)


def _gru_body(emb_ref, wir, wiz, win, whr, whz, whn, bir, biz, binn,
              bhr, bhz, bhn, wcls, bcls, out_ref):
    seq = emb_ref[...] * (1.0 / N)
    h = jnp.zeros((1, H), jnp.float32)
    for t in range(T):
        xt = seq[t:t + 1, :]
        r = jax.nn.sigmoid(xt @ wir[...] + bir[...] + h @ whr[...] + bhr[...])
        z = jax.nn.sigmoid(xt @ wiz[...] + biz[...] + h @ whz[...] + bhz[...])
        ng = jnp.tanh(xt @ win[...] + binn[...] + r * (h @ whn[...] + bhn[...]))
        h = (1.0 - z) * ng + z * h
    h = jnp.clip(h, -10.0, 10.0)
    out_ref[...] = jnp.dot(h, wcls[...],
                           preferred_element_type=jnp.float32) + bcls[...]


def _tc_gru(emb, W_ih, W_hh, b_ih, b_hh, W_cls, b_cls):
    wir, wiz, win = (W_ih[0:H].T, W_ih[H:2 * H].T, W_ih[2 * H:].T)
    whr, whz, whn = (W_hh[0:H].T, W_hh[H:2 * H].T, W_hh[2 * H:].T)
    bir, biz, binn = (b_ih[0:H][None], b_ih[H:2 * H][None], b_ih[2 * H:][None])
    bhr, bhz, bhn = (b_hh[0:H][None], b_hh[H:2 * H][None], b_hh[2 * H:][None])
    return pl.pallas_call(
        _gru_body,
        out_shape=jax.ShapeDtypeStruct((1, C), jnp.float32),
    )(emb, wir, wiz, win, whr, whz, whn, bir, biz, binn, bhr, bhz, bhn,
      W_cls, b_cls[None])


def kernel(x, edge_index, W_gcn, b_gcn, W_ih, W_hh, b_ih, b_hh, W_cls, b_cls):
    assert x.shape == (T, N, F) and edge_index.shape == (T, 2, E)
    edges4 = edge_index.reshape(T, 2, NCH, CH)
    ones = jnp.ones((CH,), jnp.float32)
    zeros1 = jnp.zeros((SEG,), jnp.float32)
    zeros2 = jnp.zeros((ZROWS, F), jnp.float32)

    deg = _sc_degree(edges4, ones, zeros1)
    deg_r = deg.reshape(400, 1000)
    x_r = x.reshape(400, 1000, F)
    dinv_r, xd_r = _tc_prep(deg_r, x_r)
    xd_flat = xd_r.reshape(T * N, F)
    accF = _sc_aggregate(edges4, xd_flat, zeros2)
    dinv3 = dinv_r.reshape(T, N, 1)
    emb = _tc_pool(accF, x, dinv3, W_gcn, b_gcn[None])
    return _tc_gru(emb, W_ih, W_hh, b_ih, b_hh, W_cls, b_cls)


# speedup vs baseline: 95.7588x; 1.0641x over previous
import functools

import jax
import jax.numpy as jnp
from jax import lax
from jax.experimental import pallas as pl
from jax.experimental.pallas import tpu as pltpu
from jax.experimental.pallas import tpu_sc as plsc

T, N, F, H, C, E = 8, 50000, 16, 32, 5, 800000
NC, NS = 2, 16
TPC = T // NC
CH = 80
NCH = E // CH
GC = 25
CPT = NCH // NS
NG = CPT // GC
ROWS_PT = N // NS
ZROWS = 625
SEG = 3128
SEG_LAST = N - (NS - 1) * SEG

@functools.cache
def _sc_mesh():
    return plsc.VectorSubcoreMesh(core_axis_name="c", subcore_axis_name="s",
                                  num_cores=NC, num_subcores=NS)


def _deg_body(edges, ones_hbm, zeros_hbm, deg_out, idx_b, ones_v, zb, deg_sh,
              sem_s):
    c = lax.axis_index("c")
    s = lax.axis_index("s")
    pltpu.sync_copy(ones_hbm, ones_v)
    pltpu.sync_copy(zeros_hbm, zb)
    for tl in range(TPC):
        t = c * TPC + tl
        @pl.when(s < NS - 1)
        def _():
            pltpu.sync_copy(zb.at[pl.ds(0, SEG)], deg_sh.at[pl.ds(s * SEG, SEG)])
        @pl.when(s == NS - 1)
        def _():
            pltpu.sync_copy(zb.at[pl.ds(0, SEG_LAST)],
                            deg_sh.at[pl.ds((NS - 1) * SEG, SEG_LAST)])
        plsc.subcore_barrier()

        def group(g, carry):
            cb = s * CPT + g * GC
            pltpu.sync_copy(edges.at[t, 1, pl.ds(cb, GC)], idx_b)
            descs = []
            for k in range(GC):
                descs.append(pltpu.async_copy(
                    ones_v, deg_sh.at[idx_b.at[k]], sem_s, add=True))
            for d in descs:
                d.wait()
            return carry
        lax.fori_loop(0, NG, group, 0)
        plsc.subcore_barrier()

        @pl.when(s < NS - 1)
        def _():
            pltpu.sync_copy(deg_sh.at[pl.ds(s * SEG, SEG)],
                            deg_out.at[t, pl.ds(s * SEG, SEG)])
        @pl.when(s == NS - 1)
        def _():
            pltpu.sync_copy(deg_sh.at[pl.ds((NS - 1) * SEG, SEG_LAST)],
                            deg_out.at[t, pl.ds((NS - 1) * SEG, SEG_LAST)])
        plsc.subcore_barrier()


def _sc_degree(*args):
    return pl.kernel(
        _deg_body,
        out_type=jax.ShapeDtypeStruct((T, N), jnp.float32),
        mesh=_sc_mesh(),
        compiler_params=pltpu.CompilerParams(use_tc_tiling_on_sc=False),
        scratch_types=[
            pltpu.VMEM((GC, CH), jnp.int32),
            pltpu.VMEM((CH,), jnp.float32),
            pltpu.VMEM((SEG,), jnp.float32),
            pltpu.VMEM_SHARED((N,), jnp.float32),
            pltpu.SemaphoreType.DMA,
        ],
    )(*args)


def _prep_body(deg_ref, x_ref, xd_ref, dinv16_ref):
    dg = deg_ref[...] + 1.0
    di = lax.rsqrt(dg)
    xd = x_ref[...] * di[..., None]
    xd_ref[...] = xd.reshape(xd_ref.shape)
    di16 = jnp.broadcast_to(di[..., None], x_ref.shape)
    dinv16_ref[...] = di16.reshape(dinv16_ref.shape)


def _tc_prep(deg_r, x_r):
    nrows, ncols = deg_r.shape
    bb = 8
    return pl.pallas_call(
        _prep_body,
        grid=(nrows // bb,),
        in_specs=[
            pl.BlockSpec((bb, ncols), lambda i: (i, 0)),
            pl.BlockSpec((bb, ncols, F), lambda i: (i, 0, 0)),
        ],
        out_specs=[
            pl.BlockSpec((bb * ncols, F), lambda i: (i, 0)),
            pl.BlockSpec((bb * ncols, F), lambda i: (i, 0)),
        ],
        out_shape=[
            jax.ShapeDtypeStruct((nrows * ncols, F), jnp.float32),
            jax.ShapeDtypeStruct((nrows * ncols, F), jnp.float32),
        ],
    )(deg_r, x_r)


def _agg_body(edges, xd_flat, acc_out, sidx, didx, rows,
              acc_sh, sem_g, sem_s):
    c = lax.axis_index("c")
    s = lax.axis_index("s")
    for tl in range(TPC):
        t = c * TPC + tl
        tN = t * N
        for z in range(ROWS_PT // ZROWS):
            r0 = s * ROWS_PT + z * ZROWS
            pltpu.sync_copy(xd_flat.at[pl.ds(tN + r0, ZROWS)],
                            acc_sh.at[pl.ds(r0, ZROWS)])
        plsc.subcore_barrier()

        def group(g, carry):
            cb = s * CPT + g * GC
            pltpu.sync_copy(edges.at[t, 0, pl.ds(cb, GC)], sidx)
            pltpu.sync_copy(edges.at[t, 1, pl.ds(cb, GC)], didx)
            for k in range(GC):
                for j in range(CH // 16):
                    v = sidx[k, pl.ds(j * 16, 16)]
                    sidx[k, pl.ds(j * 16, 16)] = v + tN
            gd = []
            for k in range(GC):
                gd.append(pltpu.async_copy(
                    xd_flat.at[sidx.at[k]], rows.at[k], sem_g))
            sd = []
            for k in range(GC):
                gd[k].wait()
                sd.append(pltpu.async_copy(
                    rows.at[k], acc_sh.at[didx.at[k]], sem_s, add=True))
            for d in sd:
                d.wait()
            return carry
        lax.fori_loop(0, NG, group, 0)
        plsc.subcore_barrier()

        for z in range(ROWS_PT // ZROWS):
            r0 = s * ROWS_PT + z * ZROWS
            pltpu.sync_copy(acc_sh.at[pl.ds(r0, ZROWS)],
                            acc_out.at[pl.ds(tN + r0, ZROWS)])
        plsc.subcore_barrier()


def _sc_aggregate(*args):
    return pl.kernel(
        _agg_body,
        out_type=jax.ShapeDtypeStruct((T * N, F), jnp.float32),
        mesh=_sc_mesh(),
        compiler_params=pltpu.CompilerParams(use_tc_tiling_on_sc=False),
        scratch_types=[
            pltpu.VMEM((GC, CH), jnp.int32),
            pltpu.VMEM((GC, CH), jnp.int32),
            pltpu.VMEM((GC, CH, F), jnp.float32),
            pltpu.VMEM_SHARED((N, F), jnp.float32),
            pltpu.SemaphoreType.DMA,
            pltpu.SemaphoreType.DMA,
        ],
    )(*args)


_POOL_BR = 10000
_POOL_NB = N // _POOL_BR


def _pool_body(acc_ref, dinv16_ref, w_ref, b_ref, out_ref):
    i = pl.program_id(0)
    z = dinv16_ref[...] * acc_ref[...]
    y = jnp.dot(z, w_ref[...], preferred_element_type=jnp.float32) + b_ref[...]
    y = jnp.maximum(y, 0.0)
    ps = jnp.sum(y, axis=0, keepdims=True)[None]

    @pl.when(i % _POOL_NB == 0)
    def _():
        out_ref[...] = ps

    @pl.when(i % _POOL_NB != 0)
    def _():
        out_ref[...] += ps


def _tc_pool(accF, dinv16, W, b2):
    return pl.pallas_call(
        _pool_body,
        grid=(T * _POOL_NB,),
        in_specs=[
            pl.BlockSpec((_POOL_BR, F), lambda i: (i, 0)),
            pl.BlockSpec((_POOL_BR, F), lambda i: (i, 0)),
            pl.BlockSpec((F, H), lambda i: (0, 0)),
            pl.BlockSpec((1, H), lambda i: (0, 0)),
        ],
        out_specs=pl.BlockSpec((1, 1, H), lambda i: (i // _POOL_NB, 0, 0)),
        out_shape=jax.ShapeDtypeStruct((T, 1, H), jnp.float32),
    )(accF, dinv16, W, b2).reshape(T, H)


def _gru_body(emb_ref, wir, wiz, win, whr, whz, whn, bir, biz, binn,
              bhr, bhz, bhn, wcls, bcls, out_ref):
    seq = emb_ref[...] * (1.0 / N)
    h = jnp.zeros((1, H), jnp.float32)
    for t in range(T):
        xt = seq[t:t + 1, :]
        r = jax.nn.sigmoid(xt @ wir[...] + bir[...] + h @ whr[...] + bhr[...])
        z = jax.nn.sigmoid(xt @ wiz[...] + biz[...] + h @ whz[...] + bhz[...])
        ng = jnp.tanh(xt @ win[...] + binn[...] + r * (h @ whn[...] + bhn[...]))
        h = (1.0 - z) * ng + z * h
    h = jnp.clip(h, -10.0, 10.0)
    out_ref[...] = jnp.dot(h, wcls[...],
                           preferred_element_type=jnp.float32) + bcls[...]


def _tc_gru(emb, W_ih, W_hh, b_ih, b_hh, W_cls, b_cls):
    wir, wiz, win = (W_ih[0:H].T, W_ih[H:2 * H].T, W_ih[2 * H:].T)
    whr, whz, whn = (W_hh[0:H].T, W_hh[H:2 * H].T, W_hh[2 * H:].T)
    bir, biz, binn = (b_ih[0:H][None], b_ih[H:2 * H][None], b_ih[2 * H:][None])
    bhr, bhz, bhn = (b_hh[0:H][None], b_hh[H:2 * H][None], b_hh[2 * H:][None])
    return pl.pallas_call(
        _gru_body,
        out_shape=jax.ShapeDtypeStruct((1, C), jnp.float32),
    )(emb, wir, wiz, win, whr, whz, whn, bir, biz, binn, bhr, bhz, bhn,
      W_cls, b_cls[None])


def kernel(x, edge_index, W_gcn, b_gcn, W_ih, W_hh, b_ih, b_hh, W_cls, b_cls):
    assert x.shape == (T, N, F) and edge_index.shape == (T, 2, E)
    edges4 = edge_index.reshape(T, 2, NCH, CH)
    ones = jnp.ones((CH,), jnp.float32)
    zeros1 = jnp.zeros((SEG,), jnp.float32)

    deg = _sc_degree(edges4, ones, zeros1)
    deg_r = deg.reshape(400, 1000)
    x_r = x.reshape(400, 1000, F)
    xd_flat, dinv16 = _tc_prep(deg_r, x_r)
    accF = _sc_aggregate(edges4, xd_flat)
    emb = _tc_pool(accF, dinv16, W_gcn, b_gcn[None])
    return _tc_gru(emb, W_ih, W_hh, b_ih, b_hh, W_cls, b_cls)


# speedup vs baseline: 101.5509x; 1.0605x over previous
import functools

import jax
import jax.numpy as jnp
from jax import lax
from jax.experimental import pallas as pl
from jax.experimental.pallas import tpu as pltpu
from jax.experimental.pallas import tpu_sc as plsc

T, N, F, H, C, E = 8, 50000, 16, 32, 5, 800000
NC, NS = 2, 16
TPC = T // NC
CH = 80
NCH = E // CH
GC = 25
CPT = NCH // NS
NG = CPT // GC
ROWS_PT = N // NS
ZROWS = 625
SEG = 3128
SEG_LAST = N - (NS - 1) * SEG

@functools.cache
def _sc_mesh():
    return plsc.VectorSubcoreMesh(core_axis_name="c", subcore_axis_name="s",
                                  num_cores=NC, num_subcores=NS)


def _deg_body(edges, ones_hbm, zeros_hbm, deg_out, idx_b, ones_v, zb, deg_sh,
              sem_s):
    c = lax.axis_index("c")
    s = lax.axis_index("s")
    pltpu.sync_copy(ones_hbm, ones_v)
    pltpu.sync_copy(zeros_hbm, zb)
    for tl in range(TPC):
        t = c * TPC + tl
        @pl.when(s < NS - 1)
        def _():
            pltpu.sync_copy(zb.at[pl.ds(0, SEG)], deg_sh.at[pl.ds(s * SEG, SEG)])
        @pl.when(s == NS - 1)
        def _():
            pltpu.sync_copy(zb.at[pl.ds(0, SEG_LAST)],
                            deg_sh.at[pl.ds((NS - 1) * SEG, SEG_LAST)])
        plsc.subcore_barrier()

        def group(g, carry):
            cb = s * CPT + g * GC
            pltpu.sync_copy(edges.at[t, 1, pl.ds(cb, GC)], idx_b)
            descs = []
            for k in range(GC):
                descs.append(pltpu.async_copy(
                    ones_v, deg_sh.at[idx_b.at[k]], sem_s, add=True))
            for d in descs:
                d.wait()
            return carry
        lax.fori_loop(0, NG, group, 0)
        plsc.subcore_barrier()

        @pl.when(s < NS - 1)
        def _():
            pltpu.sync_copy(deg_sh.at[pl.ds(s * SEG, SEG)],
                            deg_out.at[t, pl.ds(s * SEG, SEG)])
        @pl.when(s == NS - 1)
        def _():
            pltpu.sync_copy(deg_sh.at[pl.ds((NS - 1) * SEG, SEG_LAST)],
                            deg_out.at[t, pl.ds((NS - 1) * SEG, SEG_LAST)])
        plsc.subcore_barrier()


def _sc_degree(*args):
    return pl.kernel(
        _deg_body,
        out_type=jax.ShapeDtypeStruct((T, N), jnp.float32),
        mesh=_sc_mesh(),
        compiler_params=pltpu.CompilerParams(use_tc_tiling_on_sc=False),
        scratch_types=[
            pltpu.VMEM((GC, CH), jnp.int32),
            pltpu.VMEM((CH,), jnp.float32),
            pltpu.VMEM((SEG,), jnp.float32),
            pltpu.VMEM_SHARED((N,), jnp.float32),
            pltpu.SemaphoreType.DMA,
        ],
    )(*args)


def _prep_body(deg_ref, x_ref, xd_ref, dinv16_ref):
    dg = deg_ref[...] + 1.0
    di = lax.rsqrt(dg)
    xd = x_ref[...] * di[..., None]
    xd_ref[...] = xd.reshape(xd_ref.shape)
    di16 = jnp.broadcast_to(di[..., None], x_ref.shape)
    dinv16_ref[...] = di16.reshape(dinv16_ref.shape)


def _tc_prep(deg_r, x_r):
    nrows, ncols = deg_r.shape
    bb = 8
    return pl.pallas_call(
        _prep_body,
        grid=(nrows // bb,),
        in_specs=[
            pl.BlockSpec((bb, ncols), lambda i: (i, 0)),
            pl.BlockSpec((bb, ncols, F), lambda i: (i, 0, 0)),
        ],
        out_specs=[
            pl.BlockSpec((bb * ncols, F), lambda i: (i, 0)),
            pl.BlockSpec((bb * ncols, F), lambda i: (i, 0)),
        ],
        out_shape=[
            jax.ShapeDtypeStruct((nrows * ncols, F), jnp.float32),
            jax.ShapeDtypeStruct((nrows * ncols, F), jnp.float32),
        ],
    )(deg_r, x_r)


def _agg_body(edges, xd_flat, acc_out, sidx, didx, rows,
              acc_sh, sem_g, sem_s):
    c = lax.axis_index("c")
    s = lax.axis_index("s")
    for tl in range(TPC):
        t = c * TPC + tl
        tN = t * N
        for z in range(ROWS_PT // ZROWS):
            r0 = s * ROWS_PT + z * ZROWS
            pltpu.sync_copy(xd_flat.at[pl.ds(tN + r0, ZROWS)],
                            acc_sh.at[pl.ds(r0, ZROWS)])
        plsc.subcore_barrier()

        def duo(h, carry):
            gd = [None, None]
            for b in range(2):
                g = h * 2 + b
                cb = s * CPT + g * GC
                pltpu.sync_copy(edges.at[t, 0, pl.ds(cb, GC)], sidx.at[b])
                pltpu.sync_copy(edges.at[t, 1, pl.ds(cb, GC)], didx.at[b])
                for k in range(GC):
                    for j in range(CH // 16):
                        v = sidx[b, k, pl.ds(j * 16, 16)]
                        sidx[b, k, pl.ds(j * 16, 16)] = v + tN
                gd[b] = [pltpu.async_copy(
                    xd_flat.at[sidx.at[b, k]], rows.at[b, k], sem_g)
                    for k in range(GC)]
            sd = []
            for b in range(2):
                for k in range(GC):
                    gd[b][k].wait()
                    sd.append(pltpu.async_copy(
                        rows.at[b, k], acc_sh.at[didx.at[b, k]], sem_s,
                        add=True))
            for d in sd:
                d.wait()
            return carry
        lax.fori_loop(0, NG // 2, duo, 0)

        def group(g, carry):
            cb = s * CPT + g * GC
            pltpu.sync_copy(edges.at[t, 0, pl.ds(cb, GC)], sidx.at[0])
            pltpu.sync_copy(edges.at[t, 1, pl.ds(cb, GC)], didx.at[0])
            for k in range(GC):
                for j in range(CH // 16):
                    v = sidx[0, k, pl.ds(j * 16, 16)]
                    sidx[0, k, pl.ds(j * 16, 16)] = v + tN
            gd = []
            for k in range(GC):
                gd.append(pltpu.async_copy(
                    xd_flat.at[sidx.at[0, k]], rows.at[0, k], sem_g))
            sd = []
            for k in range(GC):
                gd[k].wait()
                sd.append(pltpu.async_copy(
                    rows.at[0, k], acc_sh.at[didx.at[0, k]], sem_s, add=True))
            for d in sd:
                d.wait()
            return carry
        lax.fori_loop((NG // 2) * 2, NG, group, 0)
        plsc.subcore_barrier()

        for z in range(ROWS_PT // ZROWS):
            r0 = s * ROWS_PT + z * ZROWS
            pltpu.sync_copy(acc_sh.at[pl.ds(r0, ZROWS)],
                            acc_out.at[pl.ds(tN + r0, ZROWS)])
        plsc.subcore_barrier()


def _sc_aggregate(*args):
    return pl.kernel(
        _agg_body,
        out_type=jax.ShapeDtypeStruct((T * N, F), jnp.float32),
        mesh=_sc_mesh(),
        compiler_params=pltpu.CompilerParams(use_tc_tiling_on_sc=False),
        scratch_types=[
            pltpu.VMEM((2, GC, CH), jnp.int32),
            pltpu.VMEM((2, GC, CH), jnp.int32),
            pltpu.VMEM((2, GC, CH, F), jnp.float32),
            pltpu.VMEM_SHARED((N, F), jnp.float32),
            pltpu.SemaphoreType.DMA,
            pltpu.SemaphoreType.DMA,
        ],
    )(*args)


_POOL_BR = 10000
_POOL_NB = N // _POOL_BR


def _pool_body(acc_ref, dinv16_ref, w_ref, b_ref, out_ref):
    i = pl.program_id(0)
    z = dinv16_ref[...] * acc_ref[...]
    y = jnp.dot(z, w_ref[...], preferred_element_type=jnp.float32) + b_ref[...]
    y = jnp.maximum(y, 0.0)
    ps = jnp.sum(y, axis=0, keepdims=True)[None]

    @pl.when(i % _POOL_NB == 0)
    def _():
        out_ref[...] = ps

    @pl.when(i % _POOL_NB != 0)
    def _():
        out_ref[...] += ps


def _tc_pool(accF, dinv16, W, b2):
    return pl.pallas_call(
        _pool_body,
        grid=(T * _POOL_NB,),
        in_specs=[
            pl.BlockSpec((_POOL_BR, F), lambda i: (i, 0)),
            pl.BlockSpec((_POOL_BR, F), lambda i: (i, 0)),
            pl.BlockSpec((F, H), lambda i: (0, 0)),
            pl.BlockSpec((1, H), lambda i: (0, 0)),
        ],
        out_specs=pl.BlockSpec((1, 1, H), lambda i: (i // _POOL_NB, 0, 0)),
        out_shape=jax.ShapeDtypeStruct((T, 1, H), jnp.float32),
    )(accF, dinv16, W, b2).reshape(T, H)


def _gru_body(emb_ref, wir, wiz, win, whr, whz, whn, bir, biz, binn,
              bhr, bhz, bhn, wcls, bcls, out_ref):
    seq = emb_ref[...] * (1.0 / N)
    h = jnp.zeros((1, H), jnp.float32)
    for t in range(T):
        xt = seq[t:t + 1, :]
        r = jax.nn.sigmoid(xt @ wir[...] + bir[...] + h @ whr[...] + bhr[...])
        z = jax.nn.sigmoid(xt @ wiz[...] + biz[...] + h @ whz[...] + bhz[...])
        ng = jnp.tanh(xt @ win[...] + binn[...] + r * (h @ whn[...] + bhn[...]))
        h = (1.0 - z) * ng + z * h
    h = jnp.clip(h, -10.0, 10.0)
    out_ref[...] = jnp.dot(h, wcls[...],
                           preferred_element_type=jnp.float32) + bcls[...]


def _tc_gru(emb, W_ih, W_hh, b_ih, b_hh, W_cls, b_cls):
    wir, wiz, win = (W_ih[0:H].T, W_ih[H:2 * H].T, W_ih[2 * H:].T)
    whr, whz, whn = (W_hh[0:H].T, W_hh[H:2 * H].T, W_hh[2 * H:].T)
    bir, biz, binn = (b_ih[0:H][None], b_ih[H:2 * H][None], b_ih[2 * H:][None])
    bhr, bhz, bhn = (b_hh[0:H][None], b_hh[H:2 * H][None], b_hh[2 * H:][None])
    return pl.pallas_call(
        _gru_body,
        out_shape=jax.ShapeDtypeStruct((1, C), jnp.float32),
    )(emb, wir, wiz, win, whr, whz, whn, bir, biz, binn, bhr, bhz, bhn,
      W_cls, b_cls[None])


def kernel(x, edge_index, W_gcn, b_gcn, W_ih, W_hh, b_ih, b_hh, W_cls, b_cls):
    assert x.shape == (T, N, F) and edge_index.shape == (T, 2, E)
    edges4 = edge_index.reshape(T, 2, NCH, CH)
    ones = jnp.ones((CH,), jnp.float32)
    zeros1 = jnp.zeros((SEG,), jnp.float32)

    deg = _sc_degree(edges4, ones, zeros1)
    deg_r = deg.reshape(400, 1000)
    x_r = x.reshape(400, 1000, F)
    xd_flat, dinv16 = _tc_prep(deg_r, x_r)
    accF = _sc_aggregate(edges4, xd_flat)
    emb = _tc_pool(accF, dinv16, W_gcn, b_gcn[None])
    return _tc_gru(emb, W_ih, W_hh, b_ih, b_hh, W_cls, b_cls)
